# initial kernel scaffold (unmeasured)
import jax
import jax.numpy as jnp
from jax import lax
from jax.experimental import pallas as pl
from jax.experimental.pallas import tpu as pltpu

N_DEV = 4


def kernel(A, B):
    m, _ = A.shape
    _, n = B.shape
    m_blk = m // N_DEV

    def body(a_ref, b_ref, out_ref, comm_ref, send_sems, recv_sems):
        my = lax.axis_index("i")
        left = lax.rem(my + N_DEV - 1, N_DEV)
        right = lax.rem(my + 1, N_DEV)

        barrier_sem = pltpu.get_barrier_semaphore()
        for nbr in (left, right):
            pl.semaphore_signal(
                barrier_sem, inc=1,
                device_id=(nbr,), device_id_type=pl.DeviceIdType.MESH,
            )
        pl.semaphore_wait(barrier_sem, 2)

        def partial(c):
            return jnp.dot(
                a_ref[pl.ds(c * m_blk, m_blk), :], b_ref[:, :],
                preferred_element_type=jnp.float32,
            )

        comm_ref[0, :, :] = partial(lax.rem(my + N_DEV - 1, N_DEV))

        for h in range(N_DEV - 1):
            send_slot = h % 2
            recv_slot = (h + 1) % 2
            rdma = pltpu.make_async_remote_copy(
                src_ref=comm_ref.at[send_slot],
                dst_ref=comm_ref.at[recv_slot],
                send_sem=send_sems.at[h],
                recv_sem=recv_sems.at[h],
                device_id=(right,),
                device_id_type=pl.DeviceIdType.MESH,
            )
            rdma.start()
            c = lax.rem(my + 2 * N_DEV - 2 - h, N_DEV)
            p = partial(c)
            rdma.wait()
            if h < N_DEV - 2:
                comm_ref[recv_slot, :, :] += p
            else:
                out_ref[:, :] = comm_ref[recv_slot, :, :] + p

    return pl.pallas_call(
        body,
        out_shape=jax.ShapeDtypeStruct((m_blk, n), jnp.float32),
        in_specs=[
            pl.BlockSpec(memory_space=pltpu.VMEM),
            pl.BlockSpec(memory_space=pltpu.VMEM),
        ],
        out_specs=pl.BlockSpec(memory_space=pltpu.VMEM),
        scratch_shapes=[
            pltpu.VMEM((2, m_blk, n), jnp.float32),
            pltpu.SemaphoreType.DMA((N_DEV - 1,)),
            pltpu.SemaphoreType.DMA((N_DEV - 1,)),
        ],
        compiler_params=pltpu.CompilerParams(collective_id=0),
    )(A, B)


# baseline (device time: 341682 ns/iter reference)
import jax
import jax.numpy as jnp
from jax import lax
from jax.experimental import pallas as pl
from jax.experimental.pallas import tpu as pltpu

N_DEV = 4


def kernel(A, B):
    m, k = A.shape
    _, n = B.shape
    m_blk = m // N_DEV

    def body(a_ref, b_ref, out_ref, comm_ref, a_blk_ref, copy_sem,
             send_sems, recv_sems):
        my = lax.axis_index("i")
        left = lax.rem(my + N_DEV - 1, N_DEV)
        right = lax.rem(my + 1, N_DEV)

        barrier_sem = pltpu.get_barrier_semaphore()
        for nbr in (left, right):
            pl.semaphore_signal(
                barrier_sem, inc=1,
                device_id=(nbr,), device_id_type=pl.DeviceIdType.MESH,
            )
        pl.semaphore_wait(barrier_sem, 2)

        def partial(c):
            cp = pltpu.make_async_copy(
                a_ref.at[pl.ds(c * m_blk, m_blk), :], a_blk_ref, copy_sem
            )
            cp.start()
            cp.wait()
            return jnp.dot(
                a_blk_ref[:, :], b_ref[:, :],
                preferred_element_type=jnp.float32,
            )

        comm_ref[0, :, :] = partial(lax.rem(my + N_DEV - 1, N_DEV))

        for h in range(N_DEV - 1):
            send_slot = h % 2
            recv_slot = (h + 1) % 2
            rdma = pltpu.make_async_remote_copy(
                src_ref=comm_ref.at[send_slot],
                dst_ref=comm_ref.at[recv_slot],
                send_sem=send_sems.at[h],
                recv_sem=recv_sems.at[h],
                device_id=(right,),
                device_id_type=pl.DeviceIdType.MESH,
            )
            rdma.start()
            c = lax.rem(my + 2 * N_DEV - 2 - h, N_DEV)
            p = partial(c)
            rdma.wait()
            if h < N_DEV - 2:
                comm_ref[recv_slot, :, :] += p
            else:
                out_ref[:, :] = comm_ref[recv_slot, :, :] + p

    return pl.pallas_call(
        body,
        out_shape=jax.ShapeDtypeStruct((m_blk, n), jnp.float32),
        in_specs=[
            pl.BlockSpec(memory_space=pl.ANY),
            pl.BlockSpec(memory_space=pltpu.VMEM),
        ],
        out_specs=pl.BlockSpec(memory_space=pltpu.VMEM),
        scratch_shapes=[
            pltpu.VMEM((2, m_blk, n), jnp.float32),
            pltpu.VMEM((m_blk, k), jnp.float32),
            pltpu.SemaphoreType.DMA,
            pltpu.SemaphoreType.DMA((N_DEV - 1,)),
            pltpu.SemaphoreType.DMA((N_DEV - 1,)),
        ],
        compiler_params=pltpu.CompilerParams(
            collective_id=0, vmem_limit_bytes=63 * 1024 * 1024
        ),
    )(A, B)


# device time: 191247 ns/iter; 1.7866x vs baseline; 1.7866x over previous
import jax
import jax.numpy as jnp
from jax import lax
from jax.experimental import pallas as pl
from jax.experimental.pallas import tpu as pltpu

N_DEV = 4


def kernel(A, B):
    m, k = A.shape
    _, n = B.shape
    m_blk = m // N_DEV
    m_half = m_blk // 2

    def body(a_ref, b_ref, out_ref, cw_ref, ccw_ref, a_cw_ref, a_ccw_ref,
             cp_sems, cw_send, cw_recv, ccw_send, ccw_recv):
        my = lax.axis_index("i")
        left = lax.rem(my + N_DEV - 1, N_DEV)
        right = lax.rem(my + 1, N_DEV)

        barrier_sem = pltpu.get_barrier_semaphore()
        for nbr in (left, right):
            pl.semaphore_signal(
                barrier_sem, inc=1,
                device_id=(nbr,), device_id_type=pl.DeviceIdType.MESH,
            )
        pl.semaphore_wait(barrier_sem, 2)

        def partial(c, row_off, a_stage, sem_idx):
            cp = pltpu.make_async_copy(
                a_ref.at[pl.ds(c * m_blk + row_off, m_half), :],
                a_stage, cp_sems.at[sem_idx],
            )
            cp.start()
            cp.wait()
            return jnp.dot(a_stage[:, :], b_ref[:, :],
                           preferred_element_type=jnp.float32)

        def p_cw(c):
            return partial(c, 0, a_cw_ref, 0)

        def p_ccw(c):
            return partial(c, m_half, a_ccw_ref, 1)

        cw_ref[0, :, :] = p_cw(lax.rem(my + N_DEV - 1, N_DEV))
        ccw_ref[0, :, :] = p_ccw(lax.rem(my + 1, N_DEV))

        for h in range(N_DEV - 1):
            s = h % 2
            r = (h + 1) % 2
            cw_rdma = pltpu.make_async_remote_copy(
                src_ref=cw_ref.at[s], dst_ref=cw_ref.at[r],
                send_sem=cw_send.at[h], recv_sem=cw_recv.at[h],
                device_id=(right,), device_id_type=pl.DeviceIdType.MESH,
            )
            ccw_rdma = pltpu.make_async_remote_copy(
                src_ref=ccw_ref.at[s], dst_ref=ccw_ref.at[r],
                send_sem=ccw_send.at[h], recv_sem=ccw_recv.at[h],
                device_id=(left,), device_id_type=pl.DeviceIdType.MESH,
            )
            cw_rdma.start()
            ccw_rdma.start()
            p1 = p_cw(lax.rem(my + 2 * N_DEV - 2 - h, N_DEV))
            p2 = p_ccw(lax.rem(my + 2 + h, N_DEV))
            cw_rdma.wait()
            ccw_rdma.wait()
            if h < N_DEV - 2:
                cw_ref[r, :, :] += p1
                ccw_ref[r, :, :] += p2
            else:
                out_ref[0:m_half, :] = cw_ref[r, :, :] + p1
                out_ref[m_half:m_blk, :] = ccw_ref[r, :, :] + p2

    return pl.pallas_call(
        body,
        out_shape=jax.ShapeDtypeStruct((m_blk, n), jnp.float32),
        in_specs=[
            pl.BlockSpec(memory_space=pl.ANY),
            pl.BlockSpec(memory_space=pltpu.VMEM),
        ],
        out_specs=pl.BlockSpec(memory_space=pltpu.VMEM),
        scratch_shapes=[
            pltpu.VMEM((2, m_half, n), jnp.float32),
            pltpu.VMEM((2, m_half, n), jnp.float32),
            pltpu.VMEM((m_half, k), jnp.float32),
            pltpu.VMEM((m_half, k), jnp.float32),
            pltpu.SemaphoreType.DMA((2,)),
            pltpu.SemaphoreType.DMA((N_DEV - 1,)),
            pltpu.SemaphoreType.DMA((N_DEV - 1,)),
            pltpu.SemaphoreType.DMA((N_DEV - 1,)),
            pltpu.SemaphoreType.DMA((N_DEV - 1,)),
        ],
        compiler_params=pltpu.CompilerParams(
            collective_id=0, vmem_limit_bytes=63 * 1024 * 1024
        ),
    )(A, B)


# device time: 115150 ns/iter; 2.9673x vs baseline; 1.6609x over previous
import jax
import jax.numpy as jnp
from jax import lax
from jax.experimental import pallas as pl
from jax.experimental.pallas import tpu as pltpu

N_DEV = 4


def kernel(A, B):
    m, k = A.shape
    _, n = B.shape
    m_blk = m // N_DEV
    m_half = m_blk // 2

    def body(a_ref, b_ref, out_ref, cw_ref, ccw_ref, a_cw_ref, a_ccw_ref,
             cp_sems, cw_send, cw_recv, ccw_send, ccw_recv):
        my = lax.axis_index("i")
        left = lax.rem(my + N_DEV - 1, N_DEV)
        right = lax.rem(my + 1, N_DEV)

        barrier_sem = pltpu.get_barrier_semaphore()
        for nbr in (left, right):
            pl.semaphore_signal(
                barrier_sem, inc=1,
                device_id=(nbr,), device_id_type=pl.DeviceIdType.MESH,
            )
        pl.semaphore_wait(barrier_sem, 2)

        def partial(c, row_off, a_stage, sem_idx):
            cp = pltpu.make_async_copy(
                a_ref.at[pl.ds(c * m_blk + row_off, m_half), :],
                a_stage, cp_sems.at[sem_idx],
            )
            cp.start()
            cp.wait()
            return jnp.dot(a_stage[:, :], b_ref[:, :],
                           preferred_element_type=jnp.float32)

        def p_cw(c):
            return partial(c, 0, a_cw_ref, 0)

        def p_ccw(c):
            return partial(c, m_half, a_ccw_ref, 1)

        cw_ref[0, :, :] = p_cw(lax.rem(my + N_DEV - 1, N_DEV)).astype(jnp.bfloat16)
        ccw_ref[0, :, :] = p_ccw(lax.rem(my + 1, N_DEV)).astype(jnp.bfloat16)

        for h in range(N_DEV - 1):
            s = h % 2
            r = (h + 1) % 2
            cw_rdma = pltpu.make_async_remote_copy(
                src_ref=cw_ref.at[s], dst_ref=cw_ref.at[r],
                send_sem=cw_send.at[h], recv_sem=cw_recv.at[h],
                device_id=(right,), device_id_type=pl.DeviceIdType.MESH,
            )
            ccw_rdma = pltpu.make_async_remote_copy(
                src_ref=ccw_ref.at[s], dst_ref=ccw_ref.at[r],
                send_sem=ccw_send.at[h], recv_sem=ccw_recv.at[h],
                device_id=(left,), device_id_type=pl.DeviceIdType.MESH,
            )
            cw_rdma.start()
            ccw_rdma.start()
            p1 = p_cw(lax.rem(my + 2 * N_DEV - 2 - h, N_DEV))
            p2 = p_ccw(lax.rem(my + 2 + h, N_DEV))
            cw_rdma.wait()
            ccw_rdma.wait()
            if h < N_DEV - 2:
                cw_ref[r, :, :] = (
                    cw_ref[r, :, :].astype(jnp.float32) + p1
                ).astype(jnp.bfloat16)
                ccw_ref[r, :, :] = (
                    ccw_ref[r, :, :].astype(jnp.float32) + p2
                ).astype(jnp.bfloat16)
            else:
                out_ref[0:m_half, :] = cw_ref[r, :, :].astype(jnp.float32) + p1
                out_ref[m_half:m_blk, :] = (
                    ccw_ref[r, :, :].astype(jnp.float32) + p2
                )

    return pl.pallas_call(
        body,
        out_shape=jax.ShapeDtypeStruct((m_blk, n), jnp.float32),
        in_specs=[
            pl.BlockSpec(memory_space=pl.ANY),
            pl.BlockSpec(memory_space=pltpu.VMEM),
        ],
        out_specs=pl.BlockSpec(memory_space=pltpu.VMEM),
        scratch_shapes=[
            pltpu.VMEM((2, m_half, n), jnp.bfloat16),
            pltpu.VMEM((2, m_half, n), jnp.bfloat16),
            pltpu.VMEM((m_half, k), jnp.float32),
            pltpu.VMEM((m_half, k), jnp.float32),
            pltpu.SemaphoreType.DMA((2,)),
            pltpu.SemaphoreType.DMA((N_DEV - 1,)),
            pltpu.SemaphoreType.DMA((N_DEV - 1,)),
            pltpu.SemaphoreType.DMA((N_DEV - 1,)),
            pltpu.SemaphoreType.DMA((N_DEV - 1,)),
        ],
        compiler_params=pltpu.CompilerParams(
            collective_id=0, vmem_limit_bytes=63 * 1024 * 1024
        ),
    )(A, B)


# device time: 114608 ns/iter; 2.9813x vs baseline; 1.0047x over previous
import jax
import jax.numpy as jnp
from jax import lax
from jax.experimental import pallas as pl
from jax.experimental.pallas import tpu as pltpu

N_DEV = 4


def kernel(A, B):
    m, k = A.shape
    _, n = B.shape
    m_blk = m // N_DEV
    m_half = m_blk // 2

    def body(a_ref, b_ref, out_ref, cw_ref, ccw_ref, a_cw_ref, a_ccw_ref,
             cp_sems, cw_send, cw_recv, ccw_send, ccw_recv):
        my = lax.axis_index("i")
        left = lax.rem(my + N_DEV - 1, N_DEV)
        right = lax.rem(my + 1, N_DEV)

        barrier_sem = pltpu.get_barrier_semaphore()
        for nbr in (left, right):
            pl.semaphore_signal(
                barrier_sem, inc=1,
                device_id=(nbr,), device_id_type=pl.DeviceIdType.MESH,
            )
        pl.semaphore_wait(barrier_sem, 2)

        def partial(c, row_off, a_stage, sem_idx):
            cp = pltpu.make_async_copy(
                a_ref.at[pl.ds(c * m_blk + row_off, m_half), :],
                a_stage, cp_sems.at[sem_idx],
            )
            cp.start()
            cp.wait()
            return jnp.dot(a_stage[:, :], b_ref[:, :],
                           preferred_element_type=jnp.float32)

        def p_cw(c):
            return partial(c, 0, a_cw_ref, 0)

        def p_ccw(c):
            return partial(c, m_half, a_ccw_ref, 1)

        cw_ref[0, :, :] = p_cw(lax.rem(my + N_DEV - 1, N_DEV)).astype(jnp.bfloat16)
        ccw_ref[0, :, :] = p_ccw(lax.rem(my + 1, N_DEV)).astype(jnp.bfloat16)

        for h in range(N_DEV - 1):
            s = h % 2
            r = (h + 1) % 2
            cw_rdma = pltpu.make_async_remote_copy(
                src_ref=cw_ref.at[s], dst_ref=cw_ref.at[r],
                send_sem=cw_send.at[h], recv_sem=cw_recv.at[h],
                device_id=(right,), device_id_type=pl.DeviceIdType.MESH,
            )
            ccw_rdma = pltpu.make_async_remote_copy(
                src_ref=ccw_ref.at[s], dst_ref=ccw_ref.at[r],
                send_sem=ccw_send.at[h], recv_sem=ccw_recv.at[h],
                device_id=(left,), device_id_type=pl.DeviceIdType.MESH,
            )
            cw_rdma.start()
            ccw_rdma.start()
            p1 = p_cw(lax.rem(my + 2 * N_DEV - 2 - h, N_DEV))
            p2 = p_ccw(lax.rem(my + 2 + h, N_DEV))
            if h < N_DEV - 2:
                p1b = p1.astype(jnp.bfloat16)
                p2b = p2.astype(jnp.bfloat16)
                cw_rdma.wait()
                ccw_rdma.wait()
                cw_ref[r, :, :] = cw_ref[r, :, :] + p1b
                ccw_ref[r, :, :] = ccw_ref[r, :, :] + p2b
            else:
                cw_rdma.wait()
                ccw_rdma.wait()
                out_ref[0:m_half, :] = cw_ref[r, :, :].astype(jnp.float32) + p1
                out_ref[m_half:m_blk, :] = (
                    ccw_ref[r, :, :].astype(jnp.float32) + p2
                )

    return pl.pallas_call(
        body,
        out_shape=jax.ShapeDtypeStruct((m_blk, n), jnp.float32),
        in_specs=[
            pl.BlockSpec(memory_space=pl.ANY),
            pl.BlockSpec(memory_space=pltpu.VMEM),
        ],
        out_specs=pl.BlockSpec(memory_space=pltpu.VMEM),
        scratch_shapes=[
            pltpu.VMEM((2, m_half, n), jnp.bfloat16),
            pltpu.VMEM((2, m_half, n), jnp.bfloat16),
            pltpu.VMEM((m_half, k), jnp.float32),
            pltpu.VMEM((m_half, k), jnp.float32),
            pltpu.SemaphoreType.DMA((2,)),
            pltpu.SemaphoreType.DMA((N_DEV - 1,)),
            pltpu.SemaphoreType.DMA((N_DEV - 1,)),
            pltpu.SemaphoreType.DMA((N_DEV - 1,)),
            pltpu.SemaphoreType.DMA((N_DEV - 1,)),
        ],
        compiler_params=pltpu.CompilerParams(
            collective_id=0, vmem_limit_bytes=63 * 1024 * 1024
        ),
    )(A, B)


# device time: 105605 ns/iter; 3.2355x vs baseline; 1.0853x over previous
import jax
import jax.numpy as jnp
from jax import lax
from jax.experimental import pallas as pl
from jax.experimental.pallas import tpu as pltpu

N_DEV = 4
N_SEG = 2


def kernel(A, B):
    m, k = A.shape
    _, n = B.shape
    m_blk = m // N_DEV
    m_half = m_blk // 2
    n_seg = n // N_SEG

    def body(a_ref, b_ref, out_ref, cw_ref, ccw_ref, a_cw_ref, a_ccw_ref,
             cp_sems, cw_send, cw_recv, ccw_send, ccw_recv):
        my = lax.axis_index("i")
        left = lax.rem(my + N_DEV - 1, N_DEV)
        right = lax.rem(my + 1, N_DEV)

        barrier_sem = pltpu.get_barrier_semaphore()
        for nbr in (left, right):
            pl.semaphore_signal(
                barrier_sem, inc=1,
                device_id=(nbr,), device_id_type=pl.DeviceIdType.MESH,
            )
        pl.semaphore_wait(barrier_sem, 2)

        def stage(c, row_off, a_stage, sem_idx):
            cp = pltpu.make_async_copy(
                a_ref.at[pl.ds(c * m_blk + row_off, m_half), :],
                a_stage, cp_sems.at[sem_idx],
            )
            cp.start()
            cp.wait()

        def half_dot(a_stage, j):
            return jnp.dot(
                a_stage[:, :], b_ref[:, pl.ds(j * n_seg, n_seg)],
                preferred_element_type=jnp.float32,
            )

        def full_dot(a_stage):
            return jnp.dot(a_stage[:, :], b_ref[:, :],
                           preferred_element_type=jnp.float32)

        def make_rdma(direction_ref, sems_send, sems_recv, dst, h, j):
            return pltpu.make_async_remote_copy(
                src_ref=direction_ref.at[h % 3, :, pl.ds(j * n_seg, n_seg)],
                dst_ref=direction_ref.at[(h + 1) % 3, :,
                                         pl.ds(j * n_seg, n_seg)],
                send_sem=sems_send.at[h, j],
                recv_sem=sems_recv.at[h, j],
                device_id=(dst,),
                device_id_type=pl.DeviceIdType.MESH,
            )

        cw_rdmas = {}
        ccw_rdmas = {}

        def send(h, j, cw):
            key = (h, j)
            if cw:
                cw_rdmas[key] = make_rdma(cw_ref, cw_send, cw_recv, right, h, j)
                cw_rdmas[key].start()
            else:
                ccw_rdmas[key] = make_rdma(ccw_ref, ccw_send, ccw_recv,
                                           left, h, j)
                ccw_rdmas[key].start()

        stage(lax.rem(my + N_DEV - 1, N_DEV), 0, a_cw_ref, 0)
        cw_ref[0, :, 0:n_seg] = half_dot(a_cw_ref, 0).astype(jnp.bfloat16)
        send(0, 0, cw=True)
        stage(lax.rem(my + 1, N_DEV), m_half, a_ccw_ref, 1)
        ccw_ref[0, :, 0:n_seg] = half_dot(a_ccw_ref, 0).astype(jnp.bfloat16)
        send(0, 0, cw=False)
        cw_ref[0, :, n_seg:n] = half_dot(a_cw_ref, 1).astype(jnp.bfloat16)
        send(0, 1, cw=True)
        ccw_ref[0, :, n_seg:n] = half_dot(a_ccw_ref, 1).astype(jnp.bfloat16)
        send(0, 1, cw=False)

        for h in range(N_DEV - 1):
            r = (h + 1) % 3
            c1 = lax.rem(my + 2 * N_DEV - 2 - h, N_DEV)
            c2 = lax.rem(my + 2 + h, N_DEV)
            stage(c1, 0, a_cw_ref, 0)
            p1 = full_dot(a_cw_ref)
            stage(c2, m_half, a_ccw_ref, 1)
            p2 = full_dot(a_ccw_ref)
            last = h == N_DEV - 2
            if not last:
                p1 = p1.astype(jnp.bfloat16)
                p2 = p2.astype(jnp.bfloat16)
            for j in range(N_SEG):
                cols = pl.ds(j * n_seg, n_seg)
                cw_rdmas[(h, j)].wait()
                if not last:
                    cw_ref[r, :, cols] = cw_ref[r, :, cols] + p1[:, j * n_seg:(j + 1) * n_seg]
                    send(h + 1, j, cw=True)
                else:
                    out_ref[0:m_half, cols] = (
                        cw_ref[r, :, cols].astype(jnp.float32)
                        + p1[:, j * n_seg:(j + 1) * n_seg]
                    )
                ccw_rdmas[(h, j)].wait()
                if not last:
                    ccw_ref[r, :, cols] = ccw_ref[r, :, cols] + p2[:, j * n_seg:(j + 1) * n_seg]
                    send(h + 1, j, cw=False)
                else:
                    out_ref[m_half:m_blk, cols] = (
                        ccw_ref[r, :, cols].astype(jnp.float32)
                        + p2[:, j * n_seg:(j + 1) * n_seg]
                    )

    return pl.pallas_call(
        body,
        out_shape=jax.ShapeDtypeStruct((m_blk, n), jnp.float32),
        in_specs=[
            pl.BlockSpec(memory_space=pl.ANY),
            pl.BlockSpec(memory_space=pltpu.VMEM),
        ],
        out_specs=pl.BlockSpec(memory_space=pltpu.VMEM),
        scratch_shapes=[
            pltpu.VMEM((3, m_half, n), jnp.bfloat16),
            pltpu.VMEM((3, m_half, n), jnp.bfloat16),
            pltpu.VMEM((m_half, k), jnp.float32),
            pltpu.VMEM((m_half, k), jnp.float32),
            pltpu.SemaphoreType.DMA((2,)),
            pltpu.SemaphoreType.DMA((N_DEV - 1, N_SEG)),
            pltpu.SemaphoreType.DMA((N_DEV - 1, N_SEG)),
            pltpu.SemaphoreType.DMA((N_DEV - 1, N_SEG)),
            pltpu.SemaphoreType.DMA((N_DEV - 1, N_SEG)),
        ],
        compiler_params=pltpu.CompilerParams(
            collective_id=0, vmem_limit_bytes=63 * 1024 * 1024
        ),
    )(A, B)
